# baseline (device time: 102123 ns/iter reference)
import jax
import jax.numpy as jnp
from jax import lax
from jax.experimental import pallas as pl
from jax.experimental.pallas import tpu as pltpu

N_DEV = 16
N_Q = 4
N_J = 4


def kernel(x, w_mat, scale_x, scale_w):
    m_per, k = x.shape
    _, n_per = w_mat.shape

    def body(x_ref, w_ref, sx_ref, sw_ref, out_ref, comm_ref, w8_ref,
             send_a, send_c, send_zu, send_zd, recv_sems):
        my = lax.axis_index("i")
        q = my // N_J
        j = lax.rem(my, N_J)

        def pos(qq, jj):
            return qq * N_J + lax.rem(jj + 2 * N_J, N_J)

        p_r = pos(q, j + 1)
        p_l = pos(q, j - 1)
        s_diag = pos(q, j + 2)
        z_up = lax.rem(my + N_J, N_DEV)
        z_dn = lax.rem(my + N_DEV - N_J, N_DEV)

        barrier = pltpu.get_barrier_semaphore()
        for nbr in (p_r, p_l):
            pl.semaphore_signal(barrier, inc=1, device_id=(nbr,),
                                device_id_type=pl.DeviceIdType.MESH)
        pl.semaphore_wait(barrier, 2)

        @pl.when(q < N_Q - 1)
        def _():
            pl.semaphore_signal(barrier, inc=1, device_id=(z_up,),
                                device_id_type=pl.DeviceIdType.MESH)
            pl.semaphore_wait(barrier, 1)

        @pl.when(q > 0)
        def _():
            pl.semaphore_signal(barrier, inc=1, device_id=(z_dn,),
                                device_id_type=pl.DeviceIdType.MESH)
            pl.semaphore_wait(barrier, 1)

        comm_ref[my] = x_ref[:, :].astype(jnp.float8_e5m2)
        w8_ref[:, :] = w_ref[:, :].astype(jnp.float8_e5m2)

        scale = sx_ref[0] * sw_ref[0]

        def gemm_store(s):
            acc = lax.dot_general(
                comm_ref[s], w8_ref[:, :], (((1,), (0,)), ((), ())),
                preferred_element_type=jnp.float32,
            )
            y = acc * scale
            z = jnp.clip(y, -60.0, 60.0)
            out_ref[pl.ds(s * m_per, m_per), :] = y / (1.0 + jnp.exp(-z))

        def send(s, dev, sem_arr, sem_idx):
            pltpu.make_async_remote_copy(
                src_ref=comm_ref.at[s], dst_ref=comm_ref.at[s],
                send_sem=sem_arr.at[sem_idx], recv_sem=recv_sems.at[s],
                device_id=(dev,), device_id_type=pl.DeviceIdType.MESH,
            ).start()

        def wait_recv(s):
            pltpu.make_async_remote_copy(
                src_ref=comm_ref.at[s], dst_ref=comm_ref.at[s],
                send_sem=send_a.at[0], recv_sem=recv_sems.at[s],
                device_id=(p_r,), device_id_type=pl.DeviceIdType.MESH,
            ).wait_recv()

        def z_multicast(s, hold_q, i):
            @pl.when(jnp.logical_and(hold_q <= q, q < N_Q - 1))
            def _():
                send(s, z_up, send_zu, 2 * hold_q + i)

            @pl.when(jnp.logical_and(hold_q >= q, q > 0))
            def _():
                send(s, z_dn, send_zd, 2 * hold_q + i)

        send(my, p_r, send_a, 0)
        send(my, p_l, send_a, 1)
        z_multicast(my, q, 0)
        gemm_store(my)

        wait_recv(p_l)
        send(p_l, p_r, send_a, 2)
        gemm_store(p_l)
        wait_recv(p_r)
        gemm_store(p_r)

        for d in range(1, N_Q):
            q_b = q - d
            q_a = q + d

            @pl.when(q_b >= 0)
            def _(q_b=q_b):
                qb = lax.max(q_b, 0)
                s = pos(qb, j)
                wait_recv(s)
                z_multicast(s, qb, 0)
                send(s, p_r, send_c, 2 * qb)
                send(s, p_l, send_c, 2 * qb + 1)
                gemm_store(s)

            @pl.when(q_a <= N_Q - 1)
            def _(q_a=q_a):
                qa = lax.min(q_a, N_Q - 1)
                s = pos(qa, j)
                wait_recv(s)
                z_multicast(s, qa, 0)
                send(s, p_r, send_c, 2 * qa)
                send(s, p_l, send_c, 2 * qa + 1)
                gemm_store(s)

        wait_recv(s_diag)
        z_multicast(s_diag, q, 1)
        gemm_store(s_diag)

        for d in range(1, N_Q):
            q_b = q - d
            q_a = q + d

            @pl.when(q_b >= 0)
            def _(q_b=q_b):
                qb = lax.max(q_b, 0)
                s = pos(qb, j + 2)
                wait_recv(s)
                z_multicast(s, qb, 1)
                gemm_store(s)

            @pl.when(q_a <= N_Q - 1)
            def _(q_a=q_a):
                qa = lax.min(q_a, N_Q - 1)
                s = pos(qa, j + 2)
                wait_recv(s)
                z_multicast(s, qa, 1)
                gemm_store(s)

        for qq in range(N_Q):
            @pl.when(qq != q)
            def _(qq=qq):
                s = pos(qq, j - 1)
                wait_recv(s)
                gemm_store(s)
                s2 = pos(qq, j + 1)
                wait_recv(s2)
                gemm_store(s2)

        def drain(sem_arr, idx):
            pltpu.make_async_remote_copy(
                src_ref=comm_ref.at[0], dst_ref=comm_ref.at[0],
                send_sem=sem_arr.at[idx], recv_sem=recv_sems.at[0],
                device_id=(p_r,), device_id_type=pl.DeviceIdType.MESH,
            ).wait_send()

        for i in range(3):
            drain(send_a, i)
        for qq in range(N_Q):
            @pl.when(qq != q)
            def _(qq=qq):
                drain(send_c, 2 * qq)
                drain(send_c, 2 * qq + 1)
            for i in range(2):
                @pl.when(jnp.logical_and(qq <= q, q < N_Q - 1))
                def _(qq=qq, i=i):
                    drain(send_zu, 2 * qq + i)

                @pl.when(jnp.logical_and(qq >= q, q > 0))
                def _(qq=qq, i=i):
                    drain(send_zd, 2 * qq + i)

    return pl.pallas_call(
        body,
        out_shape=jax.ShapeDtypeStruct((N_DEV * m_per, n_per), jnp.float32),
        in_specs=[
            pl.BlockSpec(memory_space=pltpu.VMEM),
            pl.BlockSpec(memory_space=pltpu.VMEM),
            pl.BlockSpec(memory_space=pltpu.SMEM),
            pl.BlockSpec(memory_space=pltpu.SMEM),
        ],
        out_specs=pl.BlockSpec(memory_space=pltpu.VMEM),
        scratch_shapes=[
            pltpu.VMEM((N_DEV, m_per, k), jnp.float8_e5m2),
            pltpu.VMEM((k, n_per), jnp.float8_e5m2),
            pltpu.SemaphoreType.DMA((3,)),
            pltpu.SemaphoreType.DMA((8,)),
            pltpu.SemaphoreType.DMA((8,)),
            pltpu.SemaphoreType.DMA((8,)),
            pltpu.SemaphoreType.DMA((N_DEV,)),
        ],
        compiler_params=pltpu.CompilerParams(collective_id=0),
    )(x, w_mat, scale_x, scale_w)


# device time: 94826 ns/iter; 1.0770x vs baseline; 1.0770x over previous
import jax
import jax.numpy as jnp
from jax import lax
from jax.experimental import pallas as pl
from jax.experimental.pallas import tpu as pltpu

N_DEV = 16
N_Q = 4
N_J = 4


def kernel(x, w_mat, scale_x, scale_w):
    m_per, k = x.shape
    _, n_per = w_mat.shape

    def body(x_ref, w_ref, sx_ref, sw_ref, out_ref, comm_ref, w8_ref,
             send_a, send_c, send_zu, send_zd, recv_sems):
        my = lax.axis_index("i")
        q = my // N_J
        j = lax.rem(my, N_J)

        def pos(qq, jj):
            return qq * N_J + lax.rem(jj + 2 * N_J, N_J)

        p_r = pos(q, j + 1)
        p_l = pos(q, j - 1)
        s_diag = pos(q, j + 2)
        z_up = lax.rem(my + N_J, N_DEV)
        z_dn = lax.rem(my + N_DEV - N_J, N_DEV)

        barrier = pltpu.get_barrier_semaphore()
        for nbr in (p_r, p_l):
            pl.semaphore_signal(barrier, inc=1, device_id=(nbr,),
                                device_id_type=pl.DeviceIdType.MESH)
        pl.semaphore_wait(barrier, 2)

        @pl.when(q < N_Q - 1)
        def _():
            pl.semaphore_signal(barrier, inc=1, device_id=(z_up,),
                                device_id_type=pl.DeviceIdType.MESH)
            pl.semaphore_wait(barrier, 1)

        @pl.when(q > 0)
        def _():
            pl.semaphore_signal(barrier, inc=1, device_id=(z_dn,),
                                device_id_type=pl.DeviceIdType.MESH)
            pl.semaphore_wait(barrier, 1)

        comm_ref[my] = x_ref[:, :].astype(jnp.float8_e5m2)
        w8_ref[:, :] = w_ref[:, :].astype(jnp.float8_e5m2)

        scale = sx_ref[0] * sw_ref[0]

        def gemm_store(s):
            acc = lax.dot_general(
                comm_ref[s], w8_ref[:, :], (((1,), (0,)), ((), ())),
                preferred_element_type=jnp.float32,
            )
            y = acc * scale
            z = jnp.clip(y, -60.0, 60.0)
            out_ref[pl.ds(s * m_per, m_per), :] = y / (1.0 + jnp.exp(-z))

        def send(s, dev, sem_arr, sem_idx):
            pltpu.make_async_remote_copy(
                src_ref=comm_ref.at[s], dst_ref=comm_ref.at[s],
                send_sem=sem_arr.at[sem_idx], recv_sem=recv_sems.at[s],
                device_id=(dev,), device_id_type=pl.DeviceIdType.MESH,
            ).start()

        def wait_recv(s):
            pltpu.make_async_remote_copy(
                src_ref=comm_ref.at[s], dst_ref=comm_ref.at[s],
                send_sem=send_a.at[0], recv_sem=recv_sems.at[s],
                device_id=(p_r,), device_id_type=pl.DeviceIdType.MESH,
            ).wait_recv()

        def z_multicast(s, hold_q, i):
            @pl.when(jnp.logical_and(hold_q <= q, q < N_Q - 1))
            def _():
                send(s, z_up, send_zu, 2 * hold_q + i)

            @pl.when(jnp.logical_and(hold_q >= q, q > 0))
            def _():
                send(s, z_dn, send_zd, 2 * hold_q + i)

        send(my, p_r, send_a, 0)
        send(my, p_l, send_a, 1)
        z_multicast(my, q, 0)
        gemm_store(my)

        m_half = m_per // 2

        def send_half(s, rows, dev, sem_idx, recv_idx):
            pltpu.make_async_remote_copy(
                src_ref=comm_ref.at[s, pl.ds(rows, m_half)],
                dst_ref=comm_ref.at[s, pl.ds(rows, m_half)],
                send_sem=send_a.at[sem_idx], recv_sem=recv_sems.at[recv_idx],
                device_id=(dev,), device_id_type=pl.DeviceIdType.MESH,
            ).start()

        wait_recv(p_l)
        send_half(p_l, 0, p_r, 2, p_l)
        gemm_store(p_l)
        wait_recv(p_r)
        send_half(p_r, m_half, p_l, 3, N_DEV)
        gemm_store(p_r)

        def wait_recv_half(rows, recv_idx):
            pltpu.make_async_remote_copy(
                src_ref=comm_ref.at[s_diag, pl.ds(rows, m_half)],
                dst_ref=comm_ref.at[s_diag, pl.ds(rows, m_half)],
                send_sem=send_a.at[0], recv_sem=recv_sems.at[recv_idx],
                device_id=(p_r,), device_id_type=pl.DeviceIdType.MESH,
            ).wait_recv()

        def pass1_step(d):
            q_b = q - d
            q_a = q + d

            @pl.when(q_b >= 0)
            def _():
                qb = lax.max(q_b, 0)
                s = pos(qb, j)
                wait_recv(s)
                z_multicast(s, qb, 0)
                send(s, p_r, send_c, 2 * qb)
                send(s, p_l, send_c, 2 * qb + 1)
                gemm_store(s)

            @pl.when(q_a <= N_Q - 1)
            def _():
                qa = lax.min(q_a, N_Q - 1)
                s = pos(qa, j)
                wait_recv(s)
                z_multicast(s, qa, 0)
                send(s, p_r, send_c, 2 * qa)
                send(s, p_l, send_c, 2 * qa + 1)
                gemm_store(s)

        pass1_step(1)

        wait_recv_half(0, s_diag)
        wait_recv_half(m_half, N_DEV)
        z_multicast(s_diag, q, 1)
        gemm_store(s_diag)

        pass1_step(2)
        pass1_step(3)

        for d in range(1, N_Q):
            q_b = q - d
            q_a = q + d

            @pl.when(q_b >= 0)
            def _(q_b=q_b):
                qb = lax.max(q_b, 0)
                s = pos(qb, j + 2)
                wait_recv(s)
                z_multicast(s, qb, 1)
                gemm_store(s)

            @pl.when(q_a <= N_Q - 1)
            def _(q_a=q_a):
                qa = lax.min(q_a, N_Q - 1)
                s = pos(qa, j + 2)
                wait_recv(s)
                z_multicast(s, qa, 1)
                gemm_store(s)

        for qq in range(N_Q):
            @pl.when(qq != q)
            def _(qq=qq):
                s = pos(qq, j - 1)
                wait_recv(s)
                gemm_store(s)
                s2 = pos(qq, j + 1)
                wait_recv(s2)
                gemm_store(s2)

        def drain(sem_arr, idx):
            pltpu.make_async_remote_copy(
                src_ref=comm_ref.at[0], dst_ref=comm_ref.at[0],
                send_sem=sem_arr.at[idx], recv_sem=recv_sems.at[0],
                device_id=(p_r,), device_id_type=pl.DeviceIdType.MESH,
            ).wait_send()

        for i in range(2):
            drain(send_a, i)
        for i in (2, 3):
            pltpu.make_async_remote_copy(
                src_ref=comm_ref.at[0, pl.ds(0, m_half)],
                dst_ref=comm_ref.at[0, pl.ds(0, m_half)],
                send_sem=send_a.at[i], recv_sem=recv_sems.at[0],
                device_id=(p_r,), device_id_type=pl.DeviceIdType.MESH,
            ).wait_send()
        for qq in range(N_Q):
            @pl.when(qq != q)
            def _(qq=qq):
                drain(send_c, 2 * qq)
                drain(send_c, 2 * qq + 1)
            for i in range(2):
                @pl.when(jnp.logical_and(qq <= q, q < N_Q - 1))
                def _(qq=qq, i=i):
                    drain(send_zu, 2 * qq + i)

                @pl.when(jnp.logical_and(qq >= q, q > 0))
                def _(qq=qq, i=i):
                    drain(send_zd, 2 * qq + i)

    return pl.pallas_call(
        body,
        out_shape=jax.ShapeDtypeStruct((N_DEV * m_per, n_per), jnp.float32),
        in_specs=[
            pl.BlockSpec(memory_space=pltpu.VMEM),
            pl.BlockSpec(memory_space=pltpu.VMEM),
            pl.BlockSpec(memory_space=pltpu.SMEM),
            pl.BlockSpec(memory_space=pltpu.SMEM),
        ],
        out_specs=pl.BlockSpec(memory_space=pltpu.VMEM),
        scratch_shapes=[
            pltpu.VMEM((N_DEV, m_per, k), jnp.float8_e5m2),
            pltpu.VMEM((k, n_per), jnp.float8_e5m2),
            pltpu.SemaphoreType.DMA((4,)),
            pltpu.SemaphoreType.DMA((8,)),
            pltpu.SemaphoreType.DMA((8,)),
            pltpu.SemaphoreType.DMA((8,)),
            pltpu.SemaphoreType.DMA((N_DEV + 1,)),
        ],
        compiler_params=pltpu.CompilerParams(collective_id=0),
    )(x, w_mat, scale_x, scale_w)


# device time: 91740 ns/iter; 1.1132x vs baseline; 1.0336x over previous
import jax
import jax.numpy as jnp
from jax import lax
from jax.experimental import pallas as pl
from jax.experimental.pallas import tpu as pltpu

N_DEV = 16
N_Q = 4
N_J = 4


def kernel(x, w_mat, scale_x, scale_w):
    m_per, k = x.shape
    _, n_per = w_mat.shape

    def body(x_ref, w_ref, sx_ref, sw_ref, out_ref, comm_ref, w8_ref,
             send_a, send_c, send_zu, send_zd, recv_sems):
        my = lax.axis_index("i")
        q = my // N_J
        j = lax.rem(my, N_J)

        def pos(qq, jj):
            return qq * N_J + lax.rem(jj + 2 * N_J, N_J)

        p_r = pos(q, j + 1)
        p_l = pos(q, j - 1)
        s_diag = pos(q, j + 2)
        z_up = lax.rem(my + N_J, N_DEV)
        z_dn = lax.rem(my + N_DEV - N_J, N_DEV)

        barrier = pltpu.get_barrier_semaphore()
        for nbr in (p_r, p_l):
            pl.semaphore_signal(barrier, inc=1, device_id=(nbr,),
                                device_id_type=pl.DeviceIdType.MESH)

        @pl.when(q < N_Q - 1)
        def _():
            pl.semaphore_signal(barrier, inc=1, device_id=(z_up,),
                                device_id_type=pl.DeviceIdType.MESH)

        @pl.when(q > 0)
        def _():
            pl.semaphore_signal(barrier, inc=1, device_id=(z_dn,),
                                device_id_type=pl.DeviceIdType.MESH)

        pl.semaphore_wait(barrier, 2)

        @pl.when(q < N_Q - 1)
        def _():
            pl.semaphore_wait(barrier, 1)

        @pl.when(q > 0)
        def _():
            pl.semaphore_wait(barrier, 1)

        comm_ref[my] = x_ref[:, :].astype(jnp.float8_e5m2)

        scale = sx_ref[0] * sw_ref[0]

        def gemm_store(s):
            acc = lax.dot_general(
                comm_ref[s], w8_ref[:, :], (((1,), (0,)), ((), ())),
                preferred_element_type=jnp.float32,
            )
            y = acc * scale
            z = jnp.clip(y, -60.0, 60.0)
            out_ref[pl.ds(s * m_per, m_per), :] = y / (1.0 + jnp.exp(-z))

        def send(s, dev, sem_arr, sem_idx):
            pltpu.make_async_remote_copy(
                src_ref=comm_ref.at[s], dst_ref=comm_ref.at[s],
                send_sem=sem_arr.at[sem_idx], recv_sem=recv_sems.at[s],
                device_id=(dev,), device_id_type=pl.DeviceIdType.MESH,
            ).start()

        def wait_recv(s):
            pltpu.make_async_remote_copy(
                src_ref=comm_ref.at[s], dst_ref=comm_ref.at[s],
                send_sem=send_a.at[0], recv_sem=recv_sems.at[s],
                device_id=(p_r,), device_id_type=pl.DeviceIdType.MESH,
            ).wait_recv()

        def z_multicast(s, hold_q, i):
            @pl.when(jnp.logical_and(hold_q <= q, q < N_Q - 1))
            def _():
                send(s, z_up, send_zu, 2 * hold_q + i)

            @pl.when(jnp.logical_and(hold_q >= q, q > 0))
            def _():
                send(s, z_dn, send_zd, 2 * hold_q + i)

        send(my, p_r, send_a, 0)
        send(my, p_l, send_a, 1)
        z_multicast(my, q, 0)
        w8_ref[:, :] = w_ref[:, :].astype(jnp.float8_e5m2)
        gemm_store(my)

        m_half = m_per // 2

        def send_half(s, rows, dev, sem_idx, recv_idx):
            pltpu.make_async_remote_copy(
                src_ref=comm_ref.at[s, pl.ds(rows, m_half)],
                dst_ref=comm_ref.at[s, pl.ds(rows, m_half)],
                send_sem=send_a.at[sem_idx], recv_sem=recv_sems.at[recv_idx],
                device_id=(dev,), device_id_type=pl.DeviceIdType.MESH,
            ).start()

        wait_recv(p_l)
        send_half(p_l, 0, p_r, 2, p_l)
        gemm_store(p_l)
        wait_recv(p_r)
        send_half(p_r, m_half, p_l, 3, N_DEV)
        gemm_store(p_r)

        def wait_recv_half(rows, recv_idx):
            pltpu.make_async_remote_copy(
                src_ref=comm_ref.at[s_diag, pl.ds(rows, m_half)],
                dst_ref=comm_ref.at[s_diag, pl.ds(rows, m_half)],
                send_sem=send_a.at[0], recv_sem=recv_sems.at[recv_idx],
                device_id=(p_r,), device_id_type=pl.DeviceIdType.MESH,
            ).wait_recv()

        def pass1_step(d):
            q_b = q - d
            q_a = q + d

            @pl.when(q_b >= 0)
            def _():
                qb = lax.max(q_b, 0)
                s = pos(qb, j)
                wait_recv(s)
                z_multicast(s, qb, 0)
                send(s, p_r, send_c, 2 * qb)
                send(s, p_l, send_c, 2 * qb + 1)
                gemm_store(s)

            @pl.when(q_a <= N_Q - 1)
            def _():
                qa = lax.min(q_a, N_Q - 1)
                s = pos(qa, j)
                wait_recv(s)
                z_multicast(s, qa, 0)
                send(s, p_r, send_c, 2 * qa)
                send(s, p_l, send_c, 2 * qa + 1)
                gemm_store(s)

        pass1_step(1)

        wait_recv_half(0, s_diag)
        wait_recv_half(m_half, N_DEV)
        z_multicast(s_diag, q, 1)
        gemm_store(s_diag)

        pass1_step(2)
        pass1_step(3)

        for d in range(1, N_Q):
            q_b = q - d
            q_a = q + d

            @pl.when(q_b >= 0)
            def _(q_b=q_b):
                qb = lax.max(q_b, 0)
                s = pos(qb, j + 2)
                wait_recv(s)
                z_multicast(s, qb, 1)
                gemm_store(s)

            @pl.when(q_a <= N_Q - 1)
            def _(q_a=q_a):
                qa = lax.min(q_a, N_Q - 1)
                s = pos(qa, j + 2)
                wait_recv(s)
                z_multicast(s, qa, 1)
                gemm_store(s)

        for qq in range(N_Q):
            @pl.when(qq != q)
            def _(qq=qq):
                s = pos(qq, j - 1)
                wait_recv(s)
                gemm_store(s)
                s2 = pos(qq, j + 1)
                wait_recv(s2)
                gemm_store(s2)

        def drain(sem_arr, idx):
            pltpu.make_async_remote_copy(
                src_ref=comm_ref.at[0], dst_ref=comm_ref.at[0],
                send_sem=sem_arr.at[idx], recv_sem=recv_sems.at[0],
                device_id=(p_r,), device_id_type=pl.DeviceIdType.MESH,
            ).wait_send()

        for i in range(2):
            drain(send_a, i)
        for i in (2, 3):
            pltpu.make_async_remote_copy(
                src_ref=comm_ref.at[0, pl.ds(0, m_half)],
                dst_ref=comm_ref.at[0, pl.ds(0, m_half)],
                send_sem=send_a.at[i], recv_sem=recv_sems.at[0],
                device_id=(p_r,), device_id_type=pl.DeviceIdType.MESH,
            ).wait_send()
        for qq in range(N_Q):
            @pl.when(qq != q)
            def _(qq=qq):
                drain(send_c, 2 * qq)
                drain(send_c, 2 * qq + 1)
            for i in range(2):
                @pl.when(jnp.logical_and(qq <= q, q < N_Q - 1))
                def _(qq=qq, i=i):
                    drain(send_zu, 2 * qq + i)

                @pl.when(jnp.logical_and(qq >= q, q > 0))
                def _(qq=qq, i=i):
                    drain(send_zd, 2 * qq + i)

    return pl.pallas_call(
        body,
        out_shape=jax.ShapeDtypeStruct((N_DEV * m_per, n_per), jnp.float32),
        in_specs=[
            pl.BlockSpec(memory_space=pltpu.VMEM),
            pl.BlockSpec(memory_space=pltpu.VMEM),
            pl.BlockSpec(memory_space=pltpu.SMEM),
            pl.BlockSpec(memory_space=pltpu.SMEM),
        ],
        out_specs=pl.BlockSpec(memory_space=pltpu.VMEM),
        scratch_shapes=[
            pltpu.VMEM((N_DEV, m_per, k), jnp.float8_e5m2),
            pltpu.VMEM((k, n_per), jnp.float8_e5m2),
            pltpu.SemaphoreType.DMA((4,)),
            pltpu.SemaphoreType.DMA((8,)),
            pltpu.SemaphoreType.DMA((8,)),
            pltpu.SemaphoreType.DMA((8,)),
            pltpu.SemaphoreType.DMA((N_DEV + 1,)),
        ],
        compiler_params=pltpu.CompilerParams(collective_id=0),
    )(x, w_mat, scale_x, scale_w)


# device time: 81432 ns/iter; 1.2541x vs baseline; 1.1266x over previous
import jax
import jax.numpy as jnp
from jax import lax
from jax.experimental import pallas as pl
from jax.experimental.pallas import tpu as pltpu

N_DEV = 16
N_Q = 4
N_J = 4


def kernel(x, w_mat, scale_x, scale_w):
    m_per, k = x.shape
    _, n_per = w_mat.shape

    def body(x_ref, w_ref, sx_ref, sw_ref, out_ref, comm_ref, w8_ref,
             send_a, send_c, send_c2, send_zu, send_zd,
             recv_sems, recv_sems2):
        my = lax.axis_index("i")
        q = my // N_J
        j = lax.rem(my, N_J)

        def pos(qq, jj):
            return qq * N_J + lax.rem(jj + 2 * N_J, N_J)

        p_r = pos(q, j + 1)
        p_l = pos(q, j - 1)
        s_diag = pos(q, j + 2)
        z_up = lax.rem(my + N_J, N_DEV)
        z_dn = lax.rem(my + N_DEV - N_J, N_DEV)

        barrier = pltpu.get_barrier_semaphore()
        for nbr in (p_r, p_l):
            pl.semaphore_signal(barrier, inc=1, device_id=(nbr,),
                                device_id_type=pl.DeviceIdType.MESH)

        @pl.when(q < N_Q - 1)
        def _():
            pl.semaphore_signal(barrier, inc=1, device_id=(z_up,),
                                device_id_type=pl.DeviceIdType.MESH)

        @pl.when(q > 0)
        def _():
            pl.semaphore_signal(barrier, inc=1, device_id=(z_dn,),
                                device_id_type=pl.DeviceIdType.MESH)

        pl.semaphore_wait(barrier, 2)

        @pl.when(q < N_Q - 1)
        def _():
            pl.semaphore_wait(barrier, 1)

        @pl.when(q > 0)
        def _():
            pl.semaphore_wait(barrier, 1)

        comm_ref[my] = x_ref[:, :].astype(jnp.float8_e5m2)

        scale = sx_ref[0] * sw_ref[0]

        def gemm_store(s):
            acc = lax.dot_general(
                comm_ref[s], w8_ref[:, :], (((1,), (0,)), ((), ())),
                preferred_element_type=jnp.float32,
            )
            y = acc * scale
            z = jnp.clip(y, -60.0, 60.0)
            out_ref[pl.ds(s * m_per, m_per), :] = y / (1.0 + jnp.exp(-z))

        def send(s, dev, sem_arr, sem_idx):
            pltpu.make_async_remote_copy(
                src_ref=comm_ref.at[s], dst_ref=comm_ref.at[s],
                send_sem=sem_arr.at[sem_idx], recv_sem=recv_sems.at[s],
                device_id=(dev,), device_id_type=pl.DeviceIdType.MESH,
            ).start()

        def wait_recv(s):
            pltpu.make_async_remote_copy(
                src_ref=comm_ref.at[s], dst_ref=comm_ref.at[s],
                send_sem=send_a.at[0], recv_sem=recv_sems.at[s],
                device_id=(p_r,), device_id_type=pl.DeviceIdType.MESH,
            ).wait_recv()

        m_half = m_per // 2
        m_qtr = m_per // 4

        def send_rows(s, r0, nrows, dev, sem_arr, sem_idx, recv_arr, recv_idx):
            pltpu.make_async_remote_copy(
                src_ref=comm_ref.at[s, pl.ds(r0, nrows)],
                dst_ref=comm_ref.at[s, pl.ds(r0, nrows)],
                send_sem=sem_arr.at[sem_idx], recv_sem=recv_arr.at[recv_idx],
                device_id=(dev,), device_id_type=pl.DeviceIdType.MESH,
            ).start()

        def wait_recv_rows(s, r0, nrows, recv_arr, recv_idx):
            pltpu.make_async_remote_copy(
                src_ref=comm_ref.at[s, pl.ds(r0, nrows)],
                dst_ref=comm_ref.at[s, pl.ds(r0, nrows)],
                send_sem=send_a.at[0], recv_sem=recv_arr.at[recv_idx],
                device_id=(p_r,), device_id_type=pl.DeviceIdType.MESH,
            ).wait_recv()

        def z_multicast(s, hold_q, i):
            nrows = m_per if i == 0 else m_half

            @pl.when(jnp.logical_and(hold_q <= q, q < N_Q - 1))
            def _():
                send_rows(s, 0, nrows, z_up, send_zu, 2 * hold_q + i,
                          recv_sems, s)

            @pl.when(jnp.logical_and(hold_q >= q, q > 0))
            def _():
                send_rows(s, 0, nrows, z_dn, send_zd, 2 * hold_q + i,
                          recv_sems, s)

        send(my, p_r, send_a, 0)
        send(my, p_l, send_a, 1)
        z_multicast(my, q, 0)
        w8_ref[:, :] = w_ref[:, :].astype(jnp.float8_e5m2)
        gemm_store(my)

        wait_recv(p_l)
        send_rows(p_l, 0, m_half, p_r, send_a, 2, recv_sems, p_l)
        gemm_store(p_l)
        wait_recv(p_r)
        send_rows(p_r, m_half, m_half, p_l, send_a, 3, recv_sems, N_DEV)
        gemm_store(p_r)

        def pass1_step(d):
            q_b = q - d
            q_a = q + d

            @pl.when(q_b >= 0)
            def _():
                qb = lax.max(q_b, 0)
                s = pos(qb, j)
                wait_recv(s)
                z_multicast(s, qb, 0)
                send(s, p_r, send_c, 2 * qb)
                send(s, p_l, send_c, 2 * qb + 1)
                gemm_store(s)

            @pl.when(q_a <= N_Q - 1)
            def _():
                qa = lax.min(q_a, N_Q - 1)
                s = pos(qa, j)
                wait_recv(s)
                z_multicast(s, qa, 0)
                send(s, p_r, send_c, 2 * qa)
                send(s, p_l, send_c, 2 * qa + 1)
                gemm_store(s)

        pass1_step(1)

        wait_recv_rows(s_diag, 0, m_half, recv_sems, s_diag)
        z_multicast(s_diag, q, 1)

        pass1_step(2)

        wait_recv_rows(s_diag, m_half, m_half, recv_sems, N_DEV)
        gemm_store(s_diag)

        pass1_step(3)

        for d in range(1, N_Q):
            q_b = q - d
            q_a = q + d

            @pl.when(q_b >= 0)
            def _(q_b=q_b):
                qb = lax.max(q_b, 0)
                s = pos(qb, j + 2)
                wait_recv_rows(s, 0, m_half, recv_sems, s)
                z_multicast(s, qb, 1)

            @pl.when(q_a <= N_Q - 1)
            def _(q_a=q_a):
                qa = lax.min(q_a, N_Q - 1)
                s = pos(qa, j + 2)
                wait_recv_rows(s, 0, m_half, recv_sems, s)
                z_multicast(s, qa, 1)

            for qq_t in (q_b, q_a):
                @pl.when(jnp.logical_and(qq_t >= 0, qq_t <= N_Q - 1))
                def _(qq_t=qq_t):
                    qq = lax.min(lax.max(qq_t, 0), N_Q - 1)
                    s = pos(qq, j - 1)
                    wait_recv(s)
                    send_rows(s, m_half + m_qtr, m_qtr, p_r,
                              send_c2, 2 * qq + 1, recv_sems2, 2 * qq + 1)
                    gemm_store(s)
                    s2 = pos(qq, j + 1)
                    wait_recv(s2)
                    send_rows(s2, m_half, m_qtr, p_l,
                              send_c2, 2 * qq, recv_sems2, 2 * qq)
                    gemm_store(s2)

        for d in range(1, N_Q):
            for qq_t in (q - d, q + d):
                @pl.when(jnp.logical_and(qq_t >= 0, qq_t <= N_Q - 1))
                def _(qq_t=qq_t):
                    qq = lax.min(lax.max(qq_t, 0), N_Q - 1)
                    s = pos(qq, j + 2)
                    wait_recv_rows(s, m_half, m_qtr, recv_sems2, 2 * qq)
                    wait_recv_rows(s, m_half + m_qtr, m_qtr,
                                   recv_sems2, 2 * qq + 1)
                    gemm_store(s)

        def drain(sem_arr, idx, nrows):
            pltpu.make_async_remote_copy(
                src_ref=comm_ref.at[0, pl.ds(0, nrows)],
                dst_ref=comm_ref.at[0, pl.ds(0, nrows)],
                send_sem=sem_arr.at[idx], recv_sem=recv_sems.at[0],
                device_id=(p_r,), device_id_type=pl.DeviceIdType.MESH,
            ).wait_send()

        for i in range(2):
            drain(send_a, i, m_per)
        for i in (2, 3):
            drain(send_a, i, m_half)
        for qq in range(N_Q):
            @pl.when(qq != q)
            def _(qq=qq):
                drain(send_c, 2 * qq, m_per)
                drain(send_c, 2 * qq + 1, m_per)
                drain(send_c2, 2 * qq, m_qtr)
                drain(send_c2, 2 * qq + 1, m_qtr)
            for i in range(2):
                @pl.when(jnp.logical_and(qq <= q, q < N_Q - 1))
                def _(qq=qq, i=i):
                    drain(send_zu, 2 * qq + i, m_per if i == 0 else m_half)

                @pl.when(jnp.logical_and(qq >= q, q > 0))
                def _(qq=qq, i=i):
                    drain(send_zd, 2 * qq + i, m_per if i == 0 else m_half)

    return pl.pallas_call(
        body,
        out_shape=jax.ShapeDtypeStruct((N_DEV * m_per, n_per), jnp.float32),
        in_specs=[
            pl.BlockSpec(memory_space=pltpu.VMEM),
            pl.BlockSpec(memory_space=pltpu.VMEM),
            pl.BlockSpec(memory_space=pltpu.SMEM),
            pl.BlockSpec(memory_space=pltpu.SMEM),
        ],
        out_specs=pl.BlockSpec(memory_space=pltpu.VMEM),
        scratch_shapes=[
            pltpu.VMEM((N_DEV, m_per, k), jnp.float8_e5m2),
            pltpu.VMEM((k, n_per), jnp.float8_e5m2),
            pltpu.SemaphoreType.DMA((4,)),
            pltpu.SemaphoreType.DMA((8,)),
            pltpu.SemaphoreType.DMA((8,)),
            pltpu.SemaphoreType.DMA((8,)),
            pltpu.SemaphoreType.DMA((8,)),
            pltpu.SemaphoreType.DMA((N_DEV + 1,)),
            pltpu.SemaphoreType.DMA((8,)),
        ],
        compiler_params=pltpu.CompilerParams(collective_id=0),
    )(x, w_mat, scale_x, scale_w)
